# baseline (speedup 1.0000x reference)
import functools

import jax
import jax.numpy as jnp
from jax import lax
from jax.experimental import pallas as pl
from jax.experimental.pallas import tpu as pltpu
from jax.experimental.pallas import tpu_sc as plsc

VOCAB = 1000000
D = 64
DP = 128
T, S = 4096, 200
B = T * S
NC = 2
NS = 16
NW = NC * NS
TPW = T // NW
NBUF = 4
AHEAD = 3
VB = 128
NFB = VOCAB // VB
VREM = VOCAB - NFB * VB
KMAX = NFB // NW + 2
L = 16

_mesh = plsc.VectorSubcoreMesh(core_axis_name="c", subcore_axis_name="s")


@functools.partial(
    pl.kernel,
    out_type=jax.ShapeDtypeStruct((VOCAB, DP), jnp.float32),
    mesh=_mesh,
    compiler_params=pltpu.CompilerParams(
        use_tc_tiling_on_sc=True, needs_layout_passes=False),
    scratch_types=[
        pltpu.VMEM((2, D, VB), jnp.float32),
        pltpu.VMEM((2, VB, DP + 1), jnp.float32),
        pltpu.VMEM((D, VREM), jnp.float32),
        pltpu.SemaphoreType.DMA,
        pltpu.SemaphoreType.DMA,
        pltpu.SemaphoreType.DMA,
        pltpu.SemaphoreType.DMA,
    ],
)
def _transpose_pad(wt_hbm, out_hbm, in_v, tr_v, rem_v, gs0, gs1, ws0, ws1):
    gsem = (gs0, gs1)
    wsem = (ws0, ws1)
    wid = lax.axis_index("s") * NC + lax.axis_index("c")
    nblk = jnp.where(wid < NFB - NW * (NFB // NW), NFB // NW + 1, NFB // NW)

    def stage(k, b):
        return pltpu.make_async_copy(
            wt_hbm.at[:, pl.ds((wid + NW * k) * VB, VB)], in_v.at[b], gsem[b])

    def flush(k, b):
        return pltpu.make_async_copy(
            tr_v.at[b, :, 0:DP],
            out_hbm.at[pl.ds((wid + NW * k) * VB, VB)], wsem[b])

    riota = [lax.iota(jnp.int32, L) + k * L for k in range(VB // L)]

    def transpose_rows(src, b, nk):
        def fgroup(fg, c):
            for fi in range(8):
                f = fg * 8 + fi
                col = jnp.full((L,), f, jnp.int32)
                vals = [src[f, pl.ds(k * L, L)] for k in range(nk)]
                for k in range(nk):
                    plsc.store_scatter(tr_v.at[b], [riota[k], col], vals[k])
            return c
        lax.fori_loop(0, D // 8, fgroup, 0)

    stage(0, 0).start()

    def step(g, carry):
        for b in range(2):
            k = 2 * g + b
            nb = 1 - b
            wk = k < nblk
            nxt = k + 1

            @pl.when(jnp.logical_and(wk,
                                     jnp.logical_and(nxt < nblk, k >= 1)))
            def _():
                flush(0, nb).wait()

            @pl.when(jnp.logical_and(wk, nxt < nblk))
            def _():
                stage(nxt, nb).start()

            @pl.when(wk)
            def _():
                stage(0, b).wait()
                transpose_rows(in_v.at[b], b, VB // L)
                flush(k, b).start()
        return carry

    lax.fori_loop(0, KMAX // 2, step, 0)
    flush(0, 0).wait()
    flush(0, 1).wait()

    @pl.when(wid == NW - 1)
    def _():
        pltpu.sync_copy(wt_hbm.at[:, pl.ds(NFB * VB, VREM)], rem_v)
        transpose_rows(rem_v, 0, VREM // L)
        pltpu.sync_copy(
            tr_v.at[0, pl.ds(0, VREM), 0:DP],
            out_hbm.at[pl.ds(NFB * VB, VREM)])


@functools.partial(
    pl.kernel,
    out_type=jax.ShapeDtypeStruct((T, S, DP), jnp.float32),
    mesh=_mesh,
    compiler_params=pltpu.CompilerParams(use_tc_tiling_on_sc=False),
    scratch_types=[
        pltpu.VMEM((TPW, S), jnp.int32),
        pltpu.VMEM((NBUF, S, D), jnp.float32),
    ]
    + [pltpu.SemaphoreType.DMA] * (2 * NBUF),
)
def _emb_lookup(table_hbm, idx_hbm, out_hbm, idx_v, rows_v, *sems):
    gsem = sems[:NBUF]
    wsem = sems[NBUF:]
    wid = lax.axis_index("s") * NC + lax.axis_index("c")
    t0 = wid * TPW
    pltpu.sync_copy(idx_hbm.at[pl.ds(t0, TPW)], idx_v)

    def gather(j, b):
        return pltpu.make_async_copy(
            table_hbm.at[idx_v.at[j]], rows_v.at[b], gsem[b])

    def write(j, b):
        return pltpu.make_async_copy(
            rows_v.at[pl.ds(b, 1)],
            out_hbm.at[pl.ds(t0 + j, 1), :, 0:D], wsem[b])

    for j in range(AHEAD):
        gather(j, j % NBUF).start()

    def group(g, carry):
        for u in range(NBUF):
            j = NBUF * g + u
            b = u
            a = j + AHEAD
            ab = (u + AHEAD) % NBUF

            @pl.when(jnp.logical_and(a < TPW, a >= NBUF))
            def _():
                write(0, ab).wait()

            @pl.when(a < TPW)
            def _():
                gather(a, ab).start()

            gather(j, b).wait()
            write(j, b).start()
        return carry

    lax.fori_loop(0, TPW // NBUF, group, 0)
    for u in range(NBUF):
        write(0, u).wait()


def kernel(input_ids, wte):
    idx2 = input_ids.astype(jnp.int32) * 2
    wtep = _transpose_pad(wte.T).reshape(2 * VOCAB, D)
    outp = _emb_lookup(wtep, idx2)
    return outp[:, :, :D]


# speedup vs baseline: 1.7856x; 1.7856x over previous
import functools

import jax
import jax.numpy as jnp
from jax import lax
from jax.experimental import pallas as pl
from jax.experimental.pallas import tpu as pltpu
from jax.experimental.pallas import tpu_sc as plsc

VOCAB = 1000000
D = 64
DP = 128
T, S = 4096, 200
B = T * S
NC = 2
NS = 16
NW = NC * NS
TPW = T // NW
NBUF = 4
AHEAD = 3

_mesh = plsc.VectorSubcoreMesh(core_axis_name="c", subcore_axis_name="s")


@functools.partial(
    pl.kernel,
    out_type=jax.ShapeDtypeStruct((T, S, DP), jnp.float32),
    mesh=_mesh,
    compiler_params=pltpu.CompilerParams(use_tc_tiling_on_sc=False),
    scratch_types=[
        pltpu.VMEM((TPW, S), jnp.int32),
        pltpu.VMEM((NBUF, S, D), jnp.float32),
    ]
    + [pltpu.SemaphoreType.DMA] * (2 * NBUF),
)
def _emb_lookup(table_hbm, idx_hbm, out_hbm, idx_v, rows_v, *sems):
    gsem = sems[:NBUF]
    wsem = sems[NBUF:]
    wid = lax.axis_index("s") * NC + lax.axis_index("c")
    t0 = wid * TPW
    pltpu.sync_copy(idx_hbm.at[pl.ds(t0, TPW)], idx_v)

    def gather(j, b):
        return pltpu.make_async_copy(
            table_hbm.at[idx_v.at[j]], rows_v.at[b], gsem[b])

    def write(j, b):
        return pltpu.make_async_copy(
            rows_v.at[pl.ds(b, 1)],
            out_hbm.at[pl.ds(t0 + j, 1), :, 0:D], wsem[b])

    for j in range(AHEAD):
        gather(j, j % NBUF).start()

    def group(g, carry):
        for u in range(NBUF):
            j = NBUF * g + u
            b = u
            a = j + AHEAD
            ab = (u + AHEAD) % NBUF

            @pl.when(jnp.logical_and(a < TPW, a >= NBUF))
            def _():
                write(0, ab).wait()

            @pl.when(a < TPW)
            def _():
                gather(a, ab).start()

            gather(j, b).wait()
            write(j, b).start()
        return carry

    lax.fori_loop(0, TPW // NBUF, group, 0)
    for u in range(NBUF):
        write(0, u).wait()


def kernel(input_ids, wte):
    idx2 = input_ids.astype(jnp.int32) * 2
    wtep = jnp.pad(wte.T, ((0, DP - D), (0, 0))).T.reshape(2 * VOCAB, D)
    outp = _emb_lookup(wtep, idx2)
    return outp[:, :, :D]


# speedup vs baseline: 1.7879x; 1.0013x over previous
import functools

import jax
import jax.numpy as jnp
from jax import lax
from jax.experimental import pallas as pl
from jax.experimental.pallas import tpu as pltpu
from jax.experimental.pallas import tpu_sc as plsc

VOCAB = 1000000
D = 64
DP = 128
T, S = 4096, 200
B = T * S
NC = 2
NS = 16
NW = NC * NS
TPW = T // NW
NBUF = 8
AHEAD = 6

_mesh = plsc.VectorSubcoreMesh(core_axis_name="c", subcore_axis_name="s")


@functools.partial(
    pl.kernel,
    out_type=jax.ShapeDtypeStruct((T, S, DP), jnp.float32),
    mesh=_mesh,
    compiler_params=pltpu.CompilerParams(use_tc_tiling_on_sc=False),
    scratch_types=[
        pltpu.VMEM((TPW, S), jnp.int32),
        pltpu.VMEM((NBUF, S, D), jnp.float32),
    ]
    + [pltpu.SemaphoreType.DMA] * (2 * NBUF),
)
def _emb_lookup(table_hbm, idx_hbm, out_hbm, idx_v, rows_v, *sems):
    gsem = sems[:NBUF]
    wsem = sems[NBUF:]
    wid = lax.axis_index("s") * NC + lax.axis_index("c")
    t0 = wid * TPW
    pltpu.sync_copy(idx_hbm.at[pl.ds(t0, TPW)], idx_v)

    def gather(j, b):
        return pltpu.make_async_copy(
            table_hbm.at[idx_v.at[j]], rows_v.at[b], gsem[b])

    def write(j, b):
        return pltpu.make_async_copy(
            rows_v.at[pl.ds(b, 1)],
            out_hbm.at[pl.ds(t0 + j, 1), :, 0:D], wsem[b])

    for j in range(AHEAD):
        gather(j, j % NBUF).start()

    def group(g, carry):
        for u in range(NBUF):
            j = NBUF * g + u
            b = u
            a = j + AHEAD
            ab = (u + AHEAD) % NBUF

            @pl.when(jnp.logical_and(a < TPW, a >= NBUF))
            def _():
                write(0, ab).wait()

            @pl.when(a < TPW)
            def _():
                gather(a, ab).start()

            gather(j, b).wait()
            write(j, b).start()
        return carry

    lax.fori_loop(0, TPW // NBUF, group, 0)
    for u in range(NBUF):
        write(0, u).wait()


def kernel(input_ids, wte):
    idx2 = input_ids.astype(jnp.int32) * 2
    wtep = jnp.pad(wte.T, ((0, DP - D), (0, 0))).T.reshape(2 * VOCAB, D)
    outp = _emb_lookup(wtep, idx2)
    return outp[:, :, :D]
